# baseline (speedup 1.0000x reference)
import functools

import jax
import jax.numpy as jnp
from jax import lax
from jax.experimental import pallas as pl
from jax.experimental.pallas import tpu as pltpu
from jax.experimental.pallas import tpu_sc as plsc

_MARGIN = 0.2
_NC, _NS, _L = 2, 16, 16
_NW = _NC * _NS
_C = 128


@functools.lru_cache(maxsize=None)
def _make_sc_kernel(T, B, W, n_chunks):
    assert n_chunks % 2 == 1 and n_chunks >= 3
    assert B % (_NS * _C) == 0
    n_per_w = n_chunks * _C
    n_pack_passes = B // (_NS * _C)
    mesh = plsc.VectorSubcoreMesh(core_axis_name="c", subcore_axis_name="s")

    @functools.partial(
        pl.kernel,
        out_type=(
            jax.ShapeDtypeStruct((_NW, _L), jnp.float32),
            jax.ShapeDtypeStruct((_NC, B, W), jnp.int32),
            jax.ShapeDtypeStruct((_NC, B, W), jnp.int32),
            jax.ShapeDtypeStruct((_NC, B, W), jnp.int32),
        ),
        mesh=mesh,
        compiler_params=pltpu.CompilerParams(needs_layout_passes=False,
                                             use_tc_tiling_on_sc=False),
        scratch_types=[
            pltpu.VMEM((2, _C), jnp.int32),
            pltpu.VMEM((2, _C), jnp.int32),
            pltpu.VMEM((2, _C), jnp.int32),
            pltpu.VMEM((2, _C, W), jnp.int32),
            pltpu.VMEM((2, _C, W), jnp.int32),
            pltpu.VMEM((2, _C, W), jnp.int32),
            pltpu.VMEM((_C, 2 * W), jnp.float32),
            pltpu.VMEM((_C, W), jnp.int32),
            pltpu.VMEM((_L,), jnp.float32),
            pltpu.SemaphoreType.DMA,
            pltpu.SemaphoreType.DMA,
        ],
    )
    def tri_loss(emb_hbm, emc_hbm, mom_hbm, ia_hbm, ip_hbm, in_hbm,
                 out_hbm, pka_hbm, pkp_hbm, pkn_hbm,
                 ia_v, ip_v, in_v, ra_v, rp_v, rn_v, pf_v, po_v, acc_v,
                 sem0, sem1):
        cid = lax.axis_index("c")
        sid = lax.axis_index("s")
        wid = sid * _NC + cid
        base_w = wid * n_per_w
        lanes = lax.iota(jnp.int32, _L)
        sems = (sem0, sem1)

        rnd_c = jnp.full((_L,), 0x7FFF, jnp.uint32)
        one_c = jnp.full((_L,), 1, jnp.uint32)
        him_c = jnp.full((_L,), 0xFFFF0000, jnp.uint32)

        def pack_pass(src_hbm, dst_hbm, p):
            r0 = sid * (n_pack_passes * _C) + p * _C
            pltpu.sync_copy(src_hbm.at[pl.ds(r0, _C)], pf_v)

            def pack_group(g, carry):
                row = g * _L + lanes
                for w in range(W):
                    widx = (lanes + w) & (W - 1)
                    vlo = plsc.load_gather(pf_v, [row, widx])
                    vhi = plsc.load_gather(pf_v, [row, widx + W])
                    bl = plsc.bitcast(vlo, jnp.uint32)
                    bh = plsc.bitcast(vhi, jnp.uint32)
                    pl_ = (bl + rnd_c + ((bl >> 16) & one_c)) >> 16
                    ph = (bh + rnd_c + ((bh >> 16) & one_c)) & him_c
                    plsc.store_scatter(po_v, [row, widx],
                                       plsc.bitcast(ph | pl_, jnp.int32))
                return carry

            lax.fori_loop(0, _C // _L, pack_group, 0)
            pltpu.sync_copy(po_v, dst_hbm.at[cid].at[pl.ds(r0, _C)])

        for src_hbm, dst_hbm in ((emb_hbm, pka_hbm), (emc_hbm, pkp_hbm),
                                 (mom_hbm, pkn_hbm)):
            def pass_body(p, carry, src_hbm=src_hbm, dst_hbm=dst_hbm):
                pack_pass(src_hbm, dst_hbm, p)
                return carry

            lax.fori_loop(0, n_pack_passes, pass_body, 0)

        plsc.subcore_barrier()

        def issue(k, b):
            base = base_w + k * _C
            pltpu.sync_copy(ia_hbm.at[pl.ds(base, _C)], ia_v.at[b])
            pltpu.sync_copy(ip_hbm.at[pl.ds(base, _C)], ip_v.at[b])
            pltpu.sync_copy(in_hbm.at[pl.ds(base, _C)], in_v.at[b])
            pltpu.make_async_copy(pka_hbm.at[cid].at[ia_v.at[b]], ra_v.at[b],
                                  sems[b]).start()
            pltpu.make_async_copy(pkp_hbm.at[cid].at[ip_v.at[b]], rp_v.at[b],
                                  sems[b]).start()
            pltpu.make_async_copy(pkn_hbm.at[cid].at[in_v.at[b]], rn_v.at[b],
                                  sems[b]).start()

        def wait(b):
            pltpu.make_async_copy(pka_hbm.at[cid].at[ia_v.at[b]], ra_v.at[b],
                                  sems[b]).wait()
            pltpu.make_async_copy(pkp_hbm.at[cid].at[ip_v.at[b]], rp_v.at[b],
                                  sems[b]).wait()
            pltpu.make_async_copy(pkn_hbm.at[cid].at[in_v.at[b]], rn_v.at[b],
                                  sems[b]).wait()

        himask = jnp.full((_L,), -0x10000, jnp.int32)

        def sqacc_halves(dw):
            lo = plsc.bitcast(dw << 16, jnp.float32)
            hi = plsc.bitcast(dw & himask, jnp.float32)
            return lo * lo, hi * hi

        def compute(k, b, acc):
            base = base_w + k * _C
            ra, rp, rn = ra_v.at[b], rp_v.at[b], rn_v.at[b]

            def group_body(g, acc):
                row = g * _L + lanes
                ap = [jnp.zeros((_L,), jnp.float32) for _ in range(4)]
                an = [jnp.zeros((_L,), jnp.float32) for _ in range(4)]
                for w in range(W):
                    widx = (lanes + w) & (W - 1)
                    va = plsc.load_gather(ra, [row, widx])
                    vp = plsc.load_gather(rp, [row, widx])
                    vn = plsc.load_gather(rn, [row, widx])
                    a16 = plsc.bitcast(va, jnp.bfloat16)
                    p16 = plsc.bitcast(vp, jnp.bfloat16)
                    n16 = plsc.bitcast(vn, jnp.bfloat16)
                    dpw = plsc.bitcast(a16 - p16, jnp.int32)
                    dnw = plsc.bitcast(a16 - n16, jnp.int32)
                    j = w & 1
                    sp_lo, sp_hi = sqacc_halves(dpw)
                    sn_lo, sn_hi = sqacc_halves(dnw)
                    ap[j] = ap[j] + sp_lo
                    ap[2 + j] = ap[2 + j] + sp_hi
                    an[j] = an[j] + sn_lo
                    an[2 + j] = an[2 + j] + sn_hi
                dd = ((ap[0] - an[0]) + (ap[1] - an[1])) + \
                     ((ap[2] - an[2]) + (ap[3] - an[3]))
                dloss = jnp.maximum(dd + _MARGIN, 0.0)
                valid = (base + row) < T
                return acc + jnp.where(valid, dloss, 0.0)

            return lax.fori_loop(0, _C // _L, group_body, acc)

        issue(0, 0)

        def pair_body(i, acc):
            k = 2 * i
            issue(k + 1, 1)
            wait(0)
            acc = compute(k, 0, acc)
            issue(k + 2, 0)
            wait(1)
            return compute(k + 1, 1, acc)

        acc = lax.fori_loop(0, (n_chunks - 1) // 2, pair_body,
                            jnp.zeros((_L,), jnp.float32))
        wait(0)
        acc = compute(n_chunks - 1, 0, acc)
        acc_v[...] = acc
        pltpu.sync_copy(acc_v, out_hbm.at[wid])

    return tri_loss


def kernel(embeddings, emc_embeddings, mom_embeddings, labels, mom_labels,
           triplets):
    T = triplets.shape[0]
    B, D = embeddings.shape
    n_chunks = -(-T // (_NW * _C))
    if n_chunks % 2 == 0:
        n_chunks += 1
    Tp = _NW * _C * n_chunks
    idx = jnp.pad(triplets, ((0, Tp - T), (0, 0)))
    f = _make_sc_kernel(T, B, D // 2, n_chunks)
    partial, _, _, _ = f(embeddings, emc_embeddings, mom_embeddings,
                         idx[:, 0], idx[:, 1], idx[:, 2])
    loss = jnp.sum(partial) / jnp.float32(T)
    return (loss, jnp.asarray(T, dtype=jnp.int32))


# speedup vs baseline: 1.3954x; 1.3954x over previous
import functools

import jax
import jax.numpy as jnp
from jax import lax
from jax.experimental import pallas as pl
from jax.experimental.pallas import tpu as pltpu
from jax.experimental.pallas import tpu_sc as plsc

_MARGIN = 0.2
_NC, _NS, _L = 2, 16, 16
_NW = _NC * _NS
_C = 128


@functools.lru_cache(maxsize=None)
def _make_sc_kernel(T, W, n_chunks):
    assert n_chunks % 2 == 1 and n_chunks >= 3
    n_per_w = n_chunks * _C
    mesh = plsc.VectorSubcoreMesh(core_axis_name="c", subcore_axis_name="s")

    @functools.partial(
        pl.kernel,
        out_type=jax.ShapeDtypeStruct((_NW, _L), jnp.float32),
        mesh=mesh,
        compiler_params=pltpu.CompilerParams(needs_layout_passes=False,
                                             use_tc_tiling_on_sc=False),
        scratch_types=[
            pltpu.VMEM((2, _C), jnp.int32),
            pltpu.VMEM((2, _C), jnp.int32),
            pltpu.VMEM((2, _C), jnp.int32),
            pltpu.VMEM((2, _C, W), jnp.int32),
            pltpu.VMEM((2, _C, W), jnp.int32),
            pltpu.VMEM((2, _C, W), jnp.int32),
            pltpu.VMEM((_L,), jnp.float32),
            pltpu.SemaphoreType.DMA,
            pltpu.SemaphoreType.DMA,
        ],
    )
    def tri_loss(emb_hbm, emc_hbm, mom_hbm, ia_hbm, ip_hbm, in_hbm, out_hbm,
                 ia_v, ip_v, in_v, ra_v, rp_v, rn_v, acc_v, sem0, sem1):
        wid = lax.axis_index("s") * _NC + lax.axis_index("c")
        base_w = wid * n_per_w
        lanes = lax.iota(jnp.int32, _L)
        sems = (sem0, sem1)

        def issue(k, b):
            base = base_w + k * _C
            pltpu.sync_copy(ia_hbm.at[pl.ds(base, _C)], ia_v.at[b])
            pltpu.sync_copy(ip_hbm.at[pl.ds(base, _C)], ip_v.at[b])
            pltpu.sync_copy(in_hbm.at[pl.ds(base, _C)], in_v.at[b])
            pltpu.make_async_copy(emb_hbm.at[ia_v.at[b]], ra_v.at[b],
                                  sems[b]).start()
            pltpu.make_async_copy(emc_hbm.at[ip_v.at[b]], rp_v.at[b],
                                  sems[b]).start()
            pltpu.make_async_copy(mom_hbm.at[in_v.at[b]], rn_v.at[b],
                                  sems[b]).start()

        def wait(b):
            pltpu.make_async_copy(emb_hbm.at[ia_v.at[b]], ra_v.at[b],
                                  sems[b]).wait()
            pltpu.make_async_copy(emc_hbm.at[ip_v.at[b]], rp_v.at[b],
                                  sems[b]).wait()
            pltpu.make_async_copy(mom_hbm.at[in_v.at[b]], rn_v.at[b],
                                  sems[b]).wait()

        himask = jnp.full((_L,), -0x10000, jnp.int32)

        def sqacc_halves(dw):
            lo = plsc.bitcast(dw << 16, jnp.float32)
            hi = plsc.bitcast(dw & himask, jnp.float32)
            return lo * lo, hi * hi

        def compute(k, b, acc):
            base = base_w + k * _C
            ra, rp, rn = ra_v.at[b], rp_v.at[b], rn_v.at[b]

            def group_body(g, acc):
                row = g * _L + lanes
                ap = [jnp.zeros((_L,), jnp.float32) for _ in range(4)]
                an = [jnp.zeros((_L,), jnp.float32) for _ in range(4)]
                for w in range(W):
                    widx = (lanes + w) & (W - 1)
                    va = plsc.load_gather(ra, [row, widx])
                    vp = plsc.load_gather(rp, [row, widx])
                    vn = plsc.load_gather(rn, [row, widx])
                    a16 = plsc.bitcast(va, jnp.bfloat16)
                    p16 = plsc.bitcast(vp, jnp.bfloat16)
                    n16 = plsc.bitcast(vn, jnp.bfloat16)
                    dpw = plsc.bitcast(a16 - p16, jnp.int32)
                    dnw = plsc.bitcast(a16 - n16, jnp.int32)
                    j = w & 1
                    sp_lo, sp_hi = sqacc_halves(dpw)
                    sn_lo, sn_hi = sqacc_halves(dnw)
                    ap[j] = ap[j] + sp_lo
                    ap[2 + j] = ap[2 + j] + sp_hi
                    an[j] = an[j] + sn_lo
                    an[2 + j] = an[2 + j] + sn_hi
                dd = ((ap[0] - an[0]) + (ap[1] - an[1])) + \
                     ((ap[2] - an[2]) + (ap[3] - an[3]))
                dloss = jnp.maximum(dd + _MARGIN, 0.0)
                valid = (base + row) < T
                return acc + jnp.where(valid, dloss, 0.0)

            return lax.fori_loop(0, _C // _L, group_body, acc)

        issue(0, 0)

        def pair_body(i, acc):
            k = 2 * i
            issue(k + 1, 1)
            wait(0)
            acc = compute(k, 0, acc)
            issue(k + 2, 0)
            wait(1)
            return compute(k + 1, 1, acc)

        acc = lax.fori_loop(0, (n_chunks - 1) // 2, pair_body,
                            jnp.zeros((_L,), jnp.float32))
        wait(0)
        acc = compute(n_chunks - 1, 0, acc)
        acc_v[...] = acc
        pltpu.sync_copy(acc_v, out_hbm.at[wid])

    return tri_loss


def _pack_bf16_pairs(table):
    d = table.shape[1]
    bits = jax.lax.bitcast_convert_type(table, jnp.uint32)

    def rnd(x):
        return x + 0x7FFF + ((x >> 16) & 1)

    lo = rnd(bits[:, : d // 2]) >> 16
    hi = rnd(bits[:, d // 2:]) & jnp.uint32(0xFFFF0000)
    return jax.lax.bitcast_convert_type(hi | lo, jnp.int32)


def kernel(embeddings, emc_embeddings, mom_embeddings, labels, mom_labels,
           triplets):
    T = triplets.shape[0]
    D = embeddings.shape[1]
    n_chunks = -(-T // (_NW * _C))
    if n_chunks % 2 == 0:
        n_chunks += 1
    Tp = _NW * _C * n_chunks
    idx = jnp.pad(triplets, ((0, Tp - T), (0, 0)))
    f = _make_sc_kernel(T, D // 2, n_chunks)
    partial = f(_pack_bf16_pairs(embeddings),
                _pack_bf16_pairs(emc_embeddings),
                _pack_bf16_pairs(mom_embeddings),
                idx[:, 0], idx[:, 1], idx[:, 2])
    loss = jnp.sum(partial) / jnp.float32(T)
    return (loss, jnp.asarray(T, dtype=jnp.int32))
